# Initial kernel scaffold; baseline (speedup 1.0000x reference)
import jax
import jax.numpy as jnp
from jax.experimental import pallas as pl


def kernel(h, W):
    raise NotImplementedError("write your pallas kernel here")


# speedup vs baseline: 1.2094x; 1.2094x over previous
import functools

import jax
import jax.numpy as jnp
from jax.experimental import pallas as pl

_D_MODEL = 4096
_N_EXP = 64
_TOP_K = 8
_ID_BIAS = 1.0
_NEG_INF = float("-inf")


def _router_block(h_ref, w_ref, sel_ref, gate_ref):
    logits = jax.lax.dot_general(
        h_ref[...],
        w_ref[...],
        dimension_numbers=(((1,), (1,)), ((), ())),
        preferred_element_type=jnp.float32,
    )
    idx = jax.lax.broadcasted_iota(jnp.int32, logits.shape, 1)
    logits = logits + jnp.where(idx == _N_EXP - 1, _ID_BIAS, 0.0)

    work = logits
    sel = jnp.zeros(logits.shape, dtype=jnp.float32)
    for _ in range(_TOP_K):
        m = jnp.max(work, axis=1, keepdims=True)
        eq = work == m
        first = jnp.min(jnp.where(eq, idx, _N_EXP), axis=1, keepdims=True)
        pick = idx == first
        sel = jnp.where(pick, 1.0, sel)
        work = jnp.where(pick, _NEG_INF, work)

    sel_ref[...] = sel
    gate_ref[...] = sel * (1.0 / _TOP_K)


@functools.partial(jax.jit, static_argnames=("block_rows",))
def _router(h, W, block_rows=512):
    n_rows = h.shape[0]
    grid = (n_rows // block_rows,)
    sel, gate = pl.pallas_call(
        _router_block,
        grid=grid,
        in_specs=[
            pl.BlockSpec((block_rows, _D_MODEL), lambda i: (i, 0)),
            pl.BlockSpec((_N_EXP, _D_MODEL), lambda i: (0, 0)),
        ],
        out_specs=[
            pl.BlockSpec((block_rows, _N_EXP), lambda i: (i, 0)),
            pl.BlockSpec((block_rows, _N_EXP), lambda i: (i, 0)),
        ],
        out_shape=[
            jax.ShapeDtypeStruct((n_rows, _N_EXP), jnp.float32),
            jax.ShapeDtypeStruct((n_rows, _N_EXP), jnp.float32),
        ],
    )(h, W)
    return sel, gate


def kernel(h, W):
    sel, gate = _router(h, W)
    return sel.astype(bool), gate


# speedup vs baseline: 1.4810x; 1.2246x over previous
import functools

import jax
import jax.numpy as jnp
from jax.experimental import pallas as pl

_D_MODEL = 4096
_N_EXP = 64
_TOP_K = 8
_ID_BIAS = 1.0
_NEG_INF = float("-inf")


def _router_block(h_ref, w_ref, sel_ref, gate_ref):
    logits = jax.lax.dot_general(
        w_ref[...],
        h_ref[...],
        dimension_numbers=(((1,), (1,)), ((), ())),
        preferred_element_type=jnp.float32,
    )
    idx = jax.lax.broadcasted_iota(jnp.int32, logits.shape, 0)
    logits = logits + jnp.where(idx == _N_EXP - 1, _ID_BIAS, 0.0)

    work = logits
    sel = jnp.zeros(logits.shape, dtype=jnp.float32)
    for _ in range(_TOP_K):
        m = jnp.max(work, axis=0, keepdims=True)
        eq = work == m
        first = jnp.min(jnp.where(eq, idx, _N_EXP), axis=0, keepdims=True)
        pick = idx == first
        sel = jnp.where(pick, 1.0, sel)
        work = jnp.where(pick, _NEG_INF, work)

    sel_t = sel.T
    sel_ref[...] = sel_t
    gate_ref[...] = sel_t * (1.0 / _TOP_K)


@functools.partial(jax.jit, static_argnames=("block_rows",))
def _router(h, W, block_rows=512):
    n_rows = h.shape[0]
    grid = (n_rows // block_rows,)
    sel, gate = pl.pallas_call(
        _router_block,
        grid=grid,
        in_specs=[
            pl.BlockSpec((block_rows, _D_MODEL), lambda i: (i, 0)),
            pl.BlockSpec((_N_EXP, _D_MODEL), lambda i: (0, 0)),
        ],
        out_specs=[
            pl.BlockSpec((block_rows, _N_EXP), lambda i: (i, 0)),
            pl.BlockSpec((block_rows, _N_EXP), lambda i: (i, 0)),
        ],
        out_shape=[
            jax.ShapeDtypeStruct((n_rows, _N_EXP), jnp.float32),
            jax.ShapeDtypeStruct((n_rows, _N_EXP), jnp.float32),
        ],
    )(h, W)
    return sel, gate


def kernel(h, W):
    sel, gate = _router(h, W)
    return sel.astype(bool), gate


# speedup vs baseline: 1.5317x; 1.0342x over previous
import functools

import jax
import jax.numpy as jnp
from jax.experimental import pallas as pl

_D_MODEL = 4096
_N_EXP = 64
_TOP_K = 8
_ID_BIAS = 1.0
_NEG_INF = float("-inf")


def _router_block(h_ref, w_ref, sel_ref, gate_ref):
    logits = jax.lax.dot_general(
        w_ref[...],
        h_ref[...],
        dimension_numbers=(((1,), (1,)), ((), ())),
        preferred_element_type=jnp.float32,
    )
    idx = jax.lax.broadcasted_iota(jnp.int32, logits.shape, 0)
    logits = logits + jnp.where(idx == _N_EXP - 1, _ID_BIAS, 0.0)

    work = logits
    sel = jnp.zeros(logits.shape, dtype=jnp.float32)
    for _ in range(_TOP_K):
        m = jnp.max(work, axis=0, keepdims=True)
        eq = work == m
        first = jnp.min(jnp.where(eq, idx, _N_EXP), axis=0, keepdims=True)
        pick = idx == first
        sel = jnp.where(pick, 1.0, sel)
        work = jnp.where(pick, _NEG_INF, work)

    sel_t = sel.T
    sel_ref[...] = sel_t
    gate_ref[...] = sel_t * (1.0 / _TOP_K)


@functools.partial(jax.jit, static_argnames=("block_rows",))
def _router(h, W, block_rows=1024):
    n_rows = h.shape[0]
    grid = (n_rows // block_rows,)
    sel, gate = pl.pallas_call(
        _router_block,
        grid=grid,
        in_specs=[
            pl.BlockSpec((block_rows, _D_MODEL), lambda i: (i, 0)),
            pl.BlockSpec((_N_EXP, _D_MODEL), lambda i: (0, 0)),
        ],
        out_specs=[
            pl.BlockSpec((block_rows, _N_EXP), lambda i: (i, 0)),
            pl.BlockSpec((block_rows, _N_EXP), lambda i: (i, 0)),
        ],
        out_shape=[
            jax.ShapeDtypeStruct((n_rows, _N_EXP), jnp.float32),
            jax.ShapeDtypeStruct((n_rows, _N_EXP), jnp.float32),
        ],
    )(h, W)
    return sel, gate


def kernel(h, W):
    sel, gate = _router(h, W)
    return sel.astype(bool), gate
